# baseline (speedup 1.0000x reference)
import jax
import jax.numpy as jnp
from jax.experimental import pallas as pl
from jax.experimental.pallas import tpu as pltpu

Q = 1024
D = 384
BK = 2000
NC = 2


def _knn_body(q_ref, v_ref, idx_out, val_out, qn_ref, best_ref, bidx_ref):
    c = pl.program_id(0)
    j = pl.program_id(1)
    nb = pl.num_programs(1)

    @pl.when(j == 0)
    def _init():
        q = q_ref[...]
        qnorm = jnp.sqrt(jnp.sum(q * q, axis=1, keepdims=True))
        qn_ref[...] = q / jnp.maximum(qnorm, 1e-12)
        best_ref[...] = jnp.full((1, Q), -jnp.inf, jnp.float32)
        bidx_ref[...] = jnp.zeros((1, Q), jnp.int32)

    v = v_ref[...]
    vnorm = jnp.sqrt(jnp.sum(v * v, axis=1, keepdims=True))
    vn = v / jnp.maximum(vnorm, 1e-12)
    sims = jax.lax.dot_general(
        vn.astype(jnp.bfloat16), qn_ref[...].astype(jnp.bfloat16),
        (((1,), (1,)), ((), ())),
        preferred_element_type=jnp.float32)
    bmax = jnp.max(sims, axis=0)[None, :]
    barg = jnp.argmax(sims, axis=0)[None, :].astype(jnp.int32)
    upd = bmax > best_ref[...]
    bidx_ref[...] = jnp.where(upd, barg + (c * nb + j) * BK, bidx_ref[...])
    best_ref[...] = jnp.where(upd, bmax, best_ref[...])

    @pl.when(j == nb - 1)
    def _fin():
        idx_out[...] = bidx_ref[...][None]
        val_out[...] = best_ref[...][None]


def _merge_body(pi_ref, pv_ref, idx_out, val_out):
    v0 = pv_ref[0]
    v1 = pv_ref[1]
    take1 = v1 > v0
    idx_out[...] = jnp.where(take1, pi_ref[1], pi_ref[0])
    val_out[...] = jnp.where(take1, v1, v0)


def kernel(v_labels, vectors):
    k = vectors.shape[0]
    nb = k // (BK * NC)
    pidx, pval = pl.pallas_call(
        _knn_body,
        grid=(NC, nb),
        in_specs=[
            pl.BlockSpec((Q, D), lambda c, j: (0, 0)),
            pl.BlockSpec((BK, D), lambda c, j: (c * (k // (BK * NC)) + j, 0)),
        ],
        out_specs=[
            pl.BlockSpec((1, 1, Q), lambda c, j: (c, 0, 0)),
            pl.BlockSpec((1, 1, Q), lambda c, j: (c, 0, 0)),
        ],
        out_shape=[
            jax.ShapeDtypeStruct((NC, 1, Q), jnp.int32),
            jax.ShapeDtypeStruct((NC, 1, Q), jnp.float32),
        ],
        scratch_shapes=[
            pltpu.VMEM((Q, D), jnp.float32),
            pltpu.VMEM((1, Q), jnp.float32),
            pltpu.VMEM((1, Q), jnp.int32),
        ],
        compiler_params=pltpu.CompilerParams(
            dimension_semantics=("parallel", "arbitrary")),
    )(v_labels, vectors)
    idx, val = pl.pallas_call(
        _merge_body,
        in_specs=[
            pl.BlockSpec((NC, 1, Q), lambda: (0, 0, 0)),
            pl.BlockSpec((NC, 1, Q), lambda: (0, 0, 0)),
        ],
        out_specs=[
            pl.BlockSpec((1, Q), lambda: (0, 0)),
            pl.BlockSpec((1, Q), lambda: (0, 0)),
        ],
        out_shape=[
            jax.ShapeDtypeStruct((1, Q), jnp.int32),
            jax.ShapeDtypeStruct((1, Q), jnp.float32),
        ],
    )(pidx, pval)
    return idx.reshape(Q), val.reshape(Q)
